# baseline (speedup 1.0000x reference)
import functools

import jax
import jax.numpy as jnp
from jax import lax
from jax.experimental import pallas as pl
from jax.experimental.pallas import tpu as pltpu
from jax.experimental.pallas import tpu_sc as plsc

N = 10000
E = 320000
F_IN = 128
H = 256
C = 32
G = 64

NC = 2
NS = 16
L = 16

EPT = E // NS
KE = 128
CH = 160
HALF = CH // 2
CHB = CH // 16
EPT_PAD = CH * KE
N_PAD = 10240
NPT = N_PAD // NS

BLK = 2048
GRID = N_PAD // BLK


def _sc_agg_body(table_hbm, idx2_hbm, ew_hbm, out_hbm,
                 vblk, cblk, buf0, buf1, acc,
                 sem_g0, sem_g1, sem_s0, sem_s1):
    c = lax.axis_index("c")
    s = lax.axis_index("s")

    zv = jnp.zeros((L,), jnp.float32)

    def _zero(i, _):
        for j in range(128 // L):
            buf0[i, pl.ds(j * L, L)] = zv
        return 0

    lax.fori_loop(0, KE, _zero, 0, unroll=False)
    for k in range(NPT // KE):
        pltpu.sync_copy(buf0,
                        acc.at[pl.ds(pl.multiple_of(s * NPT + k * KE, 8), KE)])
    plsc.subcore_barrier()

    def _scale(buf, ewoff):
        def _scale16(k, _2):
            w = cblk[pl.ds(ewoff + k * L, L)]
            for rr in range(L):
                sv = w.at[jnp.full((L,), rr, jnp.int32)].get(
                    mode="promise_in_bounds")
                r = k * L + rr
                for j in range(128 // L):
                    sl = pl.ds(j * L, L)
                    buf[r, sl] = buf[r, sl] * sv
            return 0

        lax.fori_loop(0, KE // L, _scale16, 0, unroll=False)

    def _drain(buf, sem):
        pltpu.make_async_copy(table_hbm.at[pl.ds(0, KE)], buf, sem).wait()

    pltpu.sync_copy(idx2_hbm.at[c, s, 0], vblk)
    pltpu.sync_copy(ew_hbm.at[s, 0], cblk)
    pltpu.async_copy(table_hbm.at[vblk.at[0, 0]], buf0, sem_g0)

    def _pair(t, _):
        tb = lax.rem(t, 8)
        eb = 2 * tb

        @pl.when(t > 0)
        def _w1():
            _drain(buf1, sem_s1)

        @pl.when(jnp.logical_and(tb == 0, t > 0))
        def _blk():
            pltpu.sync_copy(idx2_hbm.at[c, s, t // 8], vblk)
            pltpu.sync_copy(ew_hbm.at[s, t // 8], cblk)
            pltpu.async_copy(table_hbm.at[vblk.at[0, 0]], buf0, sem_g0)

        pltpu.async_copy(table_hbm.at[vblk.at[eb + 1, 0]], buf1, sem_g1)
        _drain(buf0, sem_g0)
        _scale(buf0, eb * KE)
        _drain(buf1, sem_g1)
        pltpu.async_copy(buf0, acc.at[vblk.at[eb, 1]], sem_s0, add=True)
        _scale(buf1, (eb + 1) * KE)
        _drain(buf0, sem_s0)

        @pl.when(jnp.logical_and(tb < 7, t < HALF - 1))
        def _g0():
            pltpu.async_copy(table_hbm.at[vblk.at[eb + 2, 0]], buf0, sem_g0)

        pltpu.async_copy(buf1, acc.at[vblk.at[eb + 1, 1]], sem_s1, add=True)
        return 0

    lax.fori_loop(0, HALF, _pair, 0, unroll=False)
    _drain(buf1, sem_s1)
    plsc.subcore_barrier()

    pltpu.sync_copy(acc.at[pl.ds(pl.multiple_of(s * NPT, 8), NPT)],
                    out_hbm.at[pl.ds(pl.multiple_of(c * N_PAD + s * NPT, 8), NPT)])


_sc_agg = functools.partial(
    pl.kernel,
    out_type=jax.ShapeDtypeStruct((NC * N_PAD, 128), jnp.float32),
    mesh=plsc.VectorSubcoreMesh(core_axis_name="c", subcore_axis_name="s"),
    scratch_types=[
        pltpu.VMEM((16, 2, KE), jnp.int32),
        pltpu.VMEM((16 * KE,), jnp.float32),
        pltpu.VMEM((KE, 128), jnp.float32),
        pltpu.VMEM((KE, 128), jnp.float32),
        pltpu.VMEM_SHARED((N_PAD, 128), jnp.float32),
        pltpu.SemaphoreType.DMA,
        pltpu.SemaphoreType.DMA,
        pltpu.SemaphoreType.DMA,
        pltpu.SemaphoreType.DMA,
    ],
)(_sc_agg_body)


def _tc1_body(x_ref, w_ref, deg0_ref, hs_ref, dinv_ref):
    deg = deg0_ref[:, 0:1] + 1.0
    dinv = lax.rsqrt(deg)
    h = jnp.dot(x_ref[...], w_ref[...], preferred_element_type=jnp.float32)
    hs = h * dinv
    hs_ref[0] = hs[:, :128]
    hs_ref[1] = hs[:, 128:]
    dinv_ref[...] = dinv


def _tc1(x, W1, degp):
    return pl.pallas_call(
        _tc1_body,
        grid=(GRID,),
        in_specs=[
            pl.BlockSpec((BLK, F_IN), lambda i: (i, 0)),
            pl.BlockSpec((F_IN, H), lambda i: (0, 0)),
            pl.BlockSpec((BLK, 128), lambda i: (i, 0)),
        ],
        out_specs=[
            pl.BlockSpec((NC, BLK, 128), lambda i: (0, i, 0)),
            pl.BlockSpec((BLK, 1), lambda i: (i, 0)),
        ],
        out_shape=[
            jax.ShapeDtypeStruct((NC, N_PAD, 128), jnp.float32),
            jax.ShapeDtypeStruct((N_PAD, 1), jnp.float32),
        ],
    )(x, W1, degp)


def _tc_mid_body(agg_ref, hs_ref, dinv_ref, b_ref, w_ref, out_ref):
    dinv = dinv_ref[...]
    h = jnp.concatenate(
        [(agg_ref[0] + hs_ref[0]) * dinv, (agg_ref[1] + hs_ref[1]) * dinv],
        axis=1) + b_ref[...]
    h = jax.nn.relu(h)
    hn = jnp.dot(h, w_ref[...], preferred_element_type=jnp.float32)
    hs = hn * dinv
    out_ref[0] = hs[:, :128]
    out_ref[1] = hs[:, 128:]


def _tc_mid(agg, hs, dinv, b, W):
    return pl.pallas_call(
        _tc_mid_body,
        grid=(GRID,),
        in_specs=[
            pl.BlockSpec((NC, BLK, 128), lambda i: (0, i, 0)),
            pl.BlockSpec((NC, BLK, 128), lambda i: (0, i, 0)),
            pl.BlockSpec((BLK, 1), lambda i: (i, 0)),
            pl.BlockSpec((1, H), lambda i: (0, 0)),
            pl.BlockSpec((H, H), lambda i: (0, 0)),
        ],
        out_specs=pl.BlockSpec((NC, BLK, 128), lambda i: (0, i, 0)),
        out_shape=jax.ShapeDtypeStruct((NC, N_PAD, 128), jnp.float32),
    )(agg, hs, dinv, b, W)


def _tc4_body(agg_ref, hs_ref, dinv_ref, b_ref, batch_ref,
              lw1_ref, lb1_ref, lw2_ref, lb2_ref, out_ref, s_acc):
    step = pl.program_id(0)

    @pl.when(step == 0)
    def _init():
        s_acc[...] = jnp.zeros((G, H + 128), jnp.float32)

    dinv = dinv_ref[...]
    h3 = jnp.concatenate(
        [(agg_ref[0] + hs_ref[0]) * dinv, (agg_ref[1] + hs_ref[1]) * dinv],
        axis=1) + b_ref[...]
    bt = batch_ref[0, 0, :]
    oh = (lax.broadcasted_iota(jnp.int32, (G, BLK), 0) == bt[None, :]
          ).astype(jnp.float32)
    ext = jnp.concatenate([h3, jnp.ones((BLK, 128), jnp.float32)], axis=1)
    s_acc[...] += jnp.dot(oh, ext, preferred_element_type=jnp.float32)

    @pl.when(step == GRID - 1)
    def _fin():
        sums = s_acc[:, :H]
        cnt = s_acc[:, H:H + 1]
        pooled = sums / jnp.maximum(cnt, 1.0)
        hh = jnp.dot(pooled, lw1_ref[...],
                     preferred_element_type=jnp.float32) + lb1_ref[...]
        hh = jax.nn.relu(hh)
        hh = jnp.dot(hh, lw2_ref[...],
                     preferred_element_type=jnp.float32) + lb2_ref[...]
        m = jnp.max(hh, axis=-1, keepdims=True)
        lse = jnp.log(jnp.sum(jnp.exp(hh - m), axis=-1, keepdims=True)) + m
        out_ref[...] = hh - lse


def _tc4(agg, hs, dinv, b3, batch3, lw1, lb1, lw2, lb2):
    return pl.pallas_call(
        _tc4_body,
        grid=(GRID,),
        in_specs=[
            pl.BlockSpec((NC, BLK, 128), lambda i: (0, i, 0)),
            pl.BlockSpec((NC, BLK, 128), lambda i: (0, i, 0)),
            pl.BlockSpec((BLK, 1), lambda i: (i, 0)),
            pl.BlockSpec((1, H), lambda i: (0, 0)),
            pl.BlockSpec((1, 1, BLK), lambda i: (i, 0, 0)),
            pl.BlockSpec((H, H), lambda i: (0, 0)),
            pl.BlockSpec((1, H), lambda i: (0, 0)),
            pl.BlockSpec((H, C), lambda i: (0, 0)),
            pl.BlockSpec((1, C), lambda i: (0, 0)),
        ],
        out_specs=pl.BlockSpec((G, C), lambda i: (0, 0)),
        out_shape=jax.ShapeDtypeStruct((G, C), jnp.float32),
        scratch_shapes=[pltpu.VMEM((G, H + 128), jnp.float32)],
    )(agg, hs, dinv, b3, batch3, lw1, lb1, lw2, lb2)


def kernel(x, edge_index, edge_attr, batch, W1, b1, W2, b2, W3, b3,
           lw1, lb1, lw2, lb2):
    row = edge_index[0].astype(jnp.int32).reshape(NS, EPT)
    col = edge_index[1].astype(jnp.int32).reshape(NS, EPT)
    ew = edge_attr.astype(jnp.float32).reshape(NS, EPT)

    pad_i = jnp.zeros((NS, EPT_PAD - EPT), jnp.int32)
    pad_f = jnp.zeros((NS, EPT_PAD - EPT), jnp.float32)
    rowp = jnp.concatenate([row, pad_i], axis=1).reshape(NS, CHB, 16, 1, KE)
    colp = jnp.concatenate([col, pad_i], axis=1).reshape(NS, CHB, 16, 1, KE)
    ewp = jnp.concatenate([ew, pad_f], axis=1).reshape(NS, CHB, 16 * KE)
    idx2 = jnp.stack(
        [jnp.concatenate([rowp, colp], axis=3),
         jnp.concatenate([rowp + N_PAD, colp], axis=3)], axis=0)

    xp = jnp.pad(x, ((0, N_PAD - N), (0, 0)))
    batchp = jnp.concatenate(
        [batch.astype(jnp.int32), jnp.full((N_PAD - N,), G, jnp.int32)])

    ones_tab = jnp.ones((NC * N_PAD, 128), jnp.float32)
    degp = _sc_agg(ones_tab, idx2, ewp)

    hs1, dinv = _tc1(xp, W1, degp)
    agg1 = _sc_agg(hs1.reshape(NC * N_PAD, 128), idx2, ewp).reshape(NC, N_PAD, 128)
    hs2 = _tc_mid(agg1, hs1, dinv, b1.reshape(1, H), W2)
    agg2 = _sc_agg(hs2.reshape(NC * N_PAD, 128), idx2, ewp).reshape(NC, N_PAD, 128)
    hs3 = _tc_mid(agg2, hs2, dinv, b2.reshape(1, H), W3)
    agg3 = _sc_agg(hs3.reshape(NC * N_PAD, 128), idx2, ewp).reshape(NC, N_PAD, 128)

    batch3 = batchp.reshape(GRID, 1, BLK)
    return _tc4(agg3, hs3, dinv, b3.reshape(1, H), batch3,
                lw1, lb1.reshape(1, H), lw2, lb2.reshape(1, C))


# speedup vs baseline: 1.0310x; 1.0310x over previous
import functools

import jax
import jax.numpy as jnp
from jax import lax
from jax.experimental import pallas as pl
from jax.experimental.pallas import tpu as pltpu
from jax.experimental.pallas import tpu_sc as plsc

N = 10000
E = 320000
F_IN = 128
H = 256
C = 32
G = 64

NC = 2
NS = 16
L = 16

EPT = E // NS
KE = 128
CH = 160
HALF = CH // 2
CHB = CH // 16
EPT_PAD = CH * KE
N_PAD = 10240
NPT = N_PAD // NS

BLK = 2048
GRID = N_PAD // BLK


def _sc_agg_body(table_hbm, idx2_hbm, ew_hbm, out_hbm,
                 vblk, cblk, buf0, buf1, acc,
                 sem_g0, sem_g1, sem_s0, sem_s1):
    c = lax.axis_index("c")
    s = lax.axis_index("s")

    zv = jnp.zeros((L,), jnp.float32)

    def _zero(i, _):
        for j in range(128 // L):
            buf0[i, pl.ds(j * L, L)] = zv
        return 0

    lax.fori_loop(0, KE, _zero, 0, unroll=False)
    for k in range(NPT // KE):
        pltpu.sync_copy(buf0,
                        acc.at[pl.ds(pl.multiple_of(s * NPT + k * KE, 8), KE)])
    plsc.subcore_barrier()

    def _scale(buf, ewoff):
        def _scale16(k, _2):
            w = cblk[pl.ds(ewoff + k * L, L)]
            for rr in range(L):
                sv = w.at[jnp.full((L,), rr, jnp.int32)].get(
                    mode="promise_in_bounds")
                r = k * L + rr
                for j in range(128 // L):
                    sl = pl.ds(j * L, L)
                    buf[r, sl] = buf[r, sl] * sv
            return 0

        lax.fori_loop(0, KE // L, _scale16, 0, unroll=False)

    def _drain(buf, sem):
        pltpu.make_async_copy(table_hbm.at[pl.ds(0, KE)], buf, sem).wait()

    pltpu.sync_copy(idx2_hbm.at[c, s, 0], vblk)
    pltpu.sync_copy(ew_hbm.at[s, 0], cblk)
    pltpu.async_copy(table_hbm.at[vblk.at[0, 0]], buf0, sem_g0)

    def _pair(t, _):
        tb = lax.rem(t, 8)
        eb = 2 * tb

        @pl.when(t > 0)
        def _w1():
            _drain(buf1, sem_s1)

        @pl.when(jnp.logical_and(tb == 0, t > 0))
        def _blk():
            pltpu.sync_copy(idx2_hbm.at[c, s, t // 8], vblk)
            pltpu.sync_copy(ew_hbm.at[s, t // 8], cblk)
            pltpu.async_copy(table_hbm.at[vblk.at[0, 0]], buf0, sem_g0)

        pltpu.async_copy(table_hbm.at[vblk.at[eb + 1, 0]], buf1, sem_g1)
        _drain(buf0, sem_g0)
        _scale(buf0, eb * KE)
        pltpu.async_copy(buf0, acc.at[vblk.at[eb, 1]], sem_s0, add=True)
        _drain(buf1, sem_g1)
        _drain(buf0, sem_s0)

        @pl.when(jnp.logical_and(tb < 7, t < HALF - 1))
        def _g0():
            pltpu.async_copy(table_hbm.at[vblk.at[eb + 2, 0]], buf0, sem_g0)

        _scale(buf1, (eb + 1) * KE)
        pltpu.async_copy(buf1, acc.at[vblk.at[eb + 1, 1]], sem_s1, add=True)
        return 0

    lax.fori_loop(0, HALF, _pair, 0, unroll=False)
    _drain(buf1, sem_s1)
    plsc.subcore_barrier()

    pltpu.sync_copy(acc.at[pl.ds(pl.multiple_of(s * NPT, 8), NPT)],
                    out_hbm.at[pl.ds(pl.multiple_of(c * N_PAD + s * NPT, 8), NPT)])


_sc_agg = functools.partial(
    pl.kernel,
    out_type=jax.ShapeDtypeStruct((NC * N_PAD, 128), jnp.float32),
    mesh=plsc.VectorSubcoreMesh(core_axis_name="c", subcore_axis_name="s"),
    scratch_types=[
        pltpu.VMEM((16, 2, KE), jnp.int32),
        pltpu.VMEM((16 * KE,), jnp.float32),
        pltpu.VMEM((KE, 128), jnp.float32),
        pltpu.VMEM((KE, 128), jnp.float32),
        pltpu.VMEM_SHARED((N_PAD, 128), jnp.float32),
        pltpu.SemaphoreType.DMA,
        pltpu.SemaphoreType.DMA,
        pltpu.SemaphoreType.DMA,
        pltpu.SemaphoreType.DMA,
    ],
)(_sc_agg_body)


def _tc1_body(x_ref, w_ref, deg0_ref, hs_ref, dinv_ref):
    deg = deg0_ref[:, 0:1] + 1.0
    dinv = lax.rsqrt(deg)
    h = jnp.dot(x_ref[...], w_ref[...], preferred_element_type=jnp.float32)
    hs = h * dinv
    hs_ref[0] = hs[:, :128]
    hs_ref[1] = hs[:, 128:]
    dinv_ref[...] = dinv


def _tc1(x, W1, degp):
    return pl.pallas_call(
        _tc1_body,
        grid=(GRID,),
        in_specs=[
            pl.BlockSpec((BLK, F_IN), lambda i: (i, 0)),
            pl.BlockSpec((F_IN, H), lambda i: (0, 0)),
            pl.BlockSpec((BLK, 128), lambda i: (i, 0)),
        ],
        out_specs=[
            pl.BlockSpec((NC, BLK, 128), lambda i: (0, i, 0)),
            pl.BlockSpec((BLK, 1), lambda i: (i, 0)),
        ],
        out_shape=[
            jax.ShapeDtypeStruct((NC, N_PAD, 128), jnp.float32),
            jax.ShapeDtypeStruct((N_PAD, 1), jnp.float32),
        ],
    )(x, W1, degp)


def _tc_mid_body(agg_ref, hs_ref, dinv_ref, b_ref, w_ref, out_ref):
    dinv = dinv_ref[...]
    h = jnp.concatenate(
        [(agg_ref[0] + hs_ref[0]) * dinv, (agg_ref[1] + hs_ref[1]) * dinv],
        axis=1) + b_ref[...]
    h = jax.nn.relu(h)
    hn = jnp.dot(h, w_ref[...], preferred_element_type=jnp.float32)
    hs = hn * dinv
    out_ref[0] = hs[:, :128]
    out_ref[1] = hs[:, 128:]


def _tc_mid(agg, hs, dinv, b, W):
    return pl.pallas_call(
        _tc_mid_body,
        grid=(GRID,),
        in_specs=[
            pl.BlockSpec((NC, BLK, 128), lambda i: (0, i, 0)),
            pl.BlockSpec((NC, BLK, 128), lambda i: (0, i, 0)),
            pl.BlockSpec((BLK, 1), lambda i: (i, 0)),
            pl.BlockSpec((1, H), lambda i: (0, 0)),
            pl.BlockSpec((H, H), lambda i: (0, 0)),
        ],
        out_specs=pl.BlockSpec((NC, BLK, 128), lambda i: (0, i, 0)),
        out_shape=jax.ShapeDtypeStruct((NC, N_PAD, 128), jnp.float32),
    )(agg, hs, dinv, b, W)


def _tc4_body(agg_ref, hs_ref, dinv_ref, b_ref, batch_ref,
              lw1_ref, lb1_ref, lw2_ref, lb2_ref, out_ref, s_acc):
    step = pl.program_id(0)

    @pl.when(step == 0)
    def _init():
        s_acc[...] = jnp.zeros((G, H + 128), jnp.float32)

    dinv = dinv_ref[...]
    h3 = jnp.concatenate(
        [(agg_ref[0] + hs_ref[0]) * dinv, (agg_ref[1] + hs_ref[1]) * dinv],
        axis=1) + b_ref[...]
    bt = batch_ref[0, 0, :]
    oh = (lax.broadcasted_iota(jnp.int32, (G, BLK), 0) == bt[None, :]
          ).astype(jnp.float32)
    ext = jnp.concatenate([h3, jnp.ones((BLK, 128), jnp.float32)], axis=1)
    s_acc[...] += jnp.dot(oh, ext, preferred_element_type=jnp.float32)

    @pl.when(step == GRID - 1)
    def _fin():
        sums = s_acc[:, :H]
        cnt = s_acc[:, H:H + 1]
        pooled = sums / jnp.maximum(cnt, 1.0)
        hh = jnp.dot(pooled, lw1_ref[...],
                     preferred_element_type=jnp.float32) + lb1_ref[...]
        hh = jax.nn.relu(hh)
        hh = jnp.dot(hh, lw2_ref[...],
                     preferred_element_type=jnp.float32) + lb2_ref[...]
        m = jnp.max(hh, axis=-1, keepdims=True)
        lse = jnp.log(jnp.sum(jnp.exp(hh - m), axis=-1, keepdims=True)) + m
        out_ref[...] = hh - lse


def _tc4(agg, hs, dinv, b3, batch3, lw1, lb1, lw2, lb2):
    return pl.pallas_call(
        _tc4_body,
        grid=(GRID,),
        in_specs=[
            pl.BlockSpec((NC, BLK, 128), lambda i: (0, i, 0)),
            pl.BlockSpec((NC, BLK, 128), lambda i: (0, i, 0)),
            pl.BlockSpec((BLK, 1), lambda i: (i, 0)),
            pl.BlockSpec((1, H), lambda i: (0, 0)),
            pl.BlockSpec((1, 1, BLK), lambda i: (i, 0, 0)),
            pl.BlockSpec((H, H), lambda i: (0, 0)),
            pl.BlockSpec((1, H), lambda i: (0, 0)),
            pl.BlockSpec((H, C), lambda i: (0, 0)),
            pl.BlockSpec((1, C), lambda i: (0, 0)),
        ],
        out_specs=pl.BlockSpec((G, C), lambda i: (0, 0)),
        out_shape=jax.ShapeDtypeStruct((G, C), jnp.float32),
        scratch_shapes=[pltpu.VMEM((G, H + 128), jnp.float32)],
    )(agg, hs, dinv, b3, batch3, lw1, lb1, lw2, lb2)


def kernel(x, edge_index, edge_attr, batch, W1, b1, W2, b2, W3, b3,
           lw1, lb1, lw2, lb2):
    row = edge_index[0].astype(jnp.int32).reshape(NS, EPT)
    col = edge_index[1].astype(jnp.int32).reshape(NS, EPT)
    ew = edge_attr.astype(jnp.float32).reshape(NS, EPT)

    pad_i = jnp.zeros((NS, EPT_PAD - EPT), jnp.int32)
    pad_f = jnp.zeros((NS, EPT_PAD - EPT), jnp.float32)
    rowp = jnp.concatenate([row, pad_i], axis=1).reshape(NS, CHB, 16, 1, KE)
    colp = jnp.concatenate([col, pad_i], axis=1).reshape(NS, CHB, 16, 1, KE)
    ewp = jnp.concatenate([ew, pad_f], axis=1).reshape(NS, CHB, 16 * KE)
    idx2 = jnp.stack(
        [jnp.concatenate([rowp, colp], axis=3),
         jnp.concatenate([rowp + N_PAD, colp], axis=3)], axis=0)

    xp = jnp.pad(x, ((0, N_PAD - N), (0, 0)))
    batchp = jnp.concatenate(
        [batch.astype(jnp.int32), jnp.full((N_PAD - N,), G, jnp.int32)])

    ones_tab = jnp.ones((NC * N_PAD, 128), jnp.float32)
    degp = _sc_agg(ones_tab, idx2, ewp)

    hs1, dinv = _tc1(xp, W1, degp)
    agg1 = _sc_agg(hs1.reshape(NC * N_PAD, 128), idx2, ewp).reshape(NC, N_PAD, 128)
    hs2 = _tc_mid(agg1, hs1, dinv, b1.reshape(1, H), W2)
    agg2 = _sc_agg(hs2.reshape(NC * N_PAD, 128), idx2, ewp).reshape(NC, N_PAD, 128)
    hs3 = _tc_mid(agg2, hs2, dinv, b2.reshape(1, H), W3)
    agg3 = _sc_agg(hs3.reshape(NC * N_PAD, 128), idx2, ewp).reshape(NC, N_PAD, 128)

    batch3 = batchp.reshape(GRID, 1, BLK)
    return _tc4(agg3, hs3, dinv, b3.reshape(1, H), batch3,
                lw1, lb1.reshape(1, H), lw2, lb2.reshape(1, C))


# speedup vs baseline: 1.3645x; 1.3235x over previous
import functools

import jax
import jax.numpy as jnp
from jax import lax
from jax.experimental import pallas as pl
from jax.experimental.pallas import tpu as pltpu
from jax.experimental.pallas import tpu_sc as plsc

N = 10000
E = 320000
F_IN = 128
H = 256
C = 32
G = 64

NC = 2
NS = 16
L = 16

EPT = E // NS
KE = 128
CH = 158
HALF = CH // 2
EPT_PAD = CH * KE
N_PAD = 10240
NPT = N_PAD // NS

BLK = 2048
GRID = N_PAD // BLK


def _sc_agg_body(table_hbm, idx2_hbm, ew_hbm, out_hbm,
                 v0, v1, c0, c1, buf0, buf1, acc,
                 sem_g0, sem_g1, sem_s0, sem_s1):
    c = lax.axis_index("c")
    s = lax.axis_index("s")

    zv = jnp.zeros((L,), jnp.float32)

    def _zero(i, _):
        for j in range(128 // L):
            buf0[i, pl.ds(j * L, L)] = zv
        return 0

    lax.fori_loop(0, KE, _zero, 0, unroll=False)
    for k in range(NPT // KE):
        pltpu.sync_copy(buf0,
                        acc.at[pl.ds(pl.multiple_of(s * NPT + k * KE, 8), KE)])
    plsc.subcore_barrier()

    def _scale(buf, ew_c):
        def _scale16(k, _2):
            w = ew_c[pl.ds(k * L, L)]
            for rr in range(L):
                sv = w.at[jnp.full((L,), rr, jnp.int32)].get(
                    mode="promise_in_bounds")
                r = k * L + rr
                for j in range(128 // L):
                    sl = pl.ds(j * L, L)
                    buf[r, sl] = buf[r, sl] * sv
            return 0

        lax.fori_loop(0, KE // L, _scale16, 0, unroll=False)

    def _drain(buf, sem):
        pltpu.make_async_copy(table_hbm.at[pl.ds(0, KE)], buf, sem).wait()

    pltpu.sync_copy(idx2_hbm.at[c, s, 0], v0)
    pltpu.sync_copy(ew_hbm.at[s, 0], c0)
    pltpu.async_copy(table_hbm.at[v0.at[0]], buf0, sem_g0)

    def _pair(t, _):
        o = 2 * t + 1
        pltpu.sync_copy(idx2_hbm.at[c, s, o], v1)
        pltpu.sync_copy(ew_hbm.at[s, o], c1)

        @pl.when(t > 0)
        def _w1():
            _drain(buf1, sem_s1)

        pltpu.async_copy(table_hbm.at[v1.at[0]], buf1, sem_g1)
        _drain(buf0, sem_g0)
        _scale(buf0, c0)
        pltpu.async_copy(buf0, acc.at[v0.at[1]], sem_s0, add=True)

        @pl.when(t < HALF - 1)
        def _c0():
            pltpu.sync_copy(idx2_hbm.at[c, s, 2 * t + 2], v0)
            pltpu.sync_copy(ew_hbm.at[s, 2 * t + 2], c0)

        _drain(buf1, sem_g1)
        _drain(buf0, sem_s0)

        @pl.when(t < HALF - 1)
        def _g0():
            pltpu.async_copy(table_hbm.at[v0.at[0]], buf0, sem_g0)

        _scale(buf1, c1)
        pltpu.async_copy(buf1, acc.at[v1.at[1]], sem_s1, add=True)
        return 0

    lax.fori_loop(0, HALF, _pair, 0, unroll=False)
    _drain(buf1, sem_s1)
    plsc.subcore_barrier()

    pltpu.sync_copy(acc.at[pl.ds(pl.multiple_of(s * NPT, 8), NPT)],
                    out_hbm.at[pl.ds(pl.multiple_of(c * N_PAD + s * NPT, 8), NPT)])


_sc_agg = functools.partial(
    pl.kernel,
    out_type=jax.ShapeDtypeStruct((NC * N_PAD, 128), jnp.float32),
    mesh=plsc.VectorSubcoreMesh(core_axis_name="c", subcore_axis_name="s"),
    scratch_types=[
        pltpu.VMEM((2, KE), jnp.int32),
        pltpu.VMEM((2, KE), jnp.int32),
        pltpu.VMEM((KE,), jnp.float32),
        pltpu.VMEM((KE,), jnp.float32),
        pltpu.VMEM((KE, 128), jnp.float32),
        pltpu.VMEM((KE, 128), jnp.float32),
        pltpu.VMEM_SHARED((N_PAD, 128), jnp.float32),
        pltpu.SemaphoreType.DMA,
        pltpu.SemaphoreType.DMA,
        pltpu.SemaphoreType.DMA,
        pltpu.SemaphoreType.DMA,
    ],
)(_sc_agg_body)


def _tc1_body(x_ref, w_ref, deg0_ref, hs_ref, dinv_ref):
    deg = deg0_ref[:, 0:1] + 1.0
    dinv = lax.rsqrt(deg)
    h = jnp.dot(x_ref[...], w_ref[...], preferred_element_type=jnp.float32)
    hs = h * dinv
    hs_ref[0] = hs[:, :128]
    hs_ref[1] = hs[:, 128:]
    dinv_ref[...] = dinv


def _tc1(x, W1, degp):
    return pl.pallas_call(
        _tc1_body,
        grid=(GRID,),
        in_specs=[
            pl.BlockSpec((BLK, F_IN), lambda i: (i, 0)),
            pl.BlockSpec((F_IN, H), lambda i: (0, 0)),
            pl.BlockSpec((BLK, 128), lambda i: (i, 0)),
        ],
        out_specs=[
            pl.BlockSpec((NC, BLK, 128), lambda i: (0, i, 0)),
            pl.BlockSpec((BLK, 1), lambda i: (i, 0)),
        ],
        out_shape=[
            jax.ShapeDtypeStruct((NC, N_PAD, 128), jnp.float32),
            jax.ShapeDtypeStruct((N_PAD, 1), jnp.float32),
        ],
    )(x, W1, degp)


def _tc_mid_body(agg_ref, hs_ref, dinv_ref, b_ref, w_ref, out_ref):
    dinv = dinv_ref[...]
    h = jnp.concatenate(
        [(agg_ref[0] + hs_ref[0]) * dinv, (agg_ref[1] + hs_ref[1]) * dinv],
        axis=1) + b_ref[...]
    h = jax.nn.relu(h)
    hn = jnp.dot(h, w_ref[...], preferred_element_type=jnp.float32)
    hs = hn * dinv
    out_ref[0] = hs[:, :128]
    out_ref[1] = hs[:, 128:]


def _tc_mid(agg, hs, dinv, b, W):
    return pl.pallas_call(
        _tc_mid_body,
        grid=(GRID,),
        in_specs=[
            pl.BlockSpec((NC, BLK, 128), lambda i: (0, i, 0)),
            pl.BlockSpec((NC, BLK, 128), lambda i: (0, i, 0)),
            pl.BlockSpec((BLK, 1), lambda i: (i, 0)),
            pl.BlockSpec((1, H), lambda i: (0, 0)),
            pl.BlockSpec((H, H), lambda i: (0, 0)),
        ],
        out_specs=pl.BlockSpec((NC, BLK, 128), lambda i: (0, i, 0)),
        out_shape=jax.ShapeDtypeStruct((NC, N_PAD, 128), jnp.float32),
    )(agg, hs, dinv, b, W)


def _tc4_body(agg_ref, hs_ref, dinv_ref, b_ref, batch_ref,
              lw1_ref, lb1_ref, lw2_ref, lb2_ref, out_ref, s_acc):
    step = pl.program_id(0)

    @pl.when(step == 0)
    def _init():
        s_acc[...] = jnp.zeros((G, H + 128), jnp.float32)

    dinv = dinv_ref[...]
    h3 = jnp.concatenate(
        [(agg_ref[0] + hs_ref[0]) * dinv, (agg_ref[1] + hs_ref[1]) * dinv],
        axis=1) + b_ref[...]
    bt = batch_ref[0, 0, :]
    oh = (lax.broadcasted_iota(jnp.int32, (G, BLK), 0) == bt[None, :]
          ).astype(jnp.float32)
    ext = jnp.concatenate([h3, jnp.ones((BLK, 128), jnp.float32)], axis=1)
    s_acc[...] += jnp.dot(oh, ext, preferred_element_type=jnp.float32)

    @pl.when(step == GRID - 1)
    def _fin():
        sums = s_acc[:, :H]
        cnt = s_acc[:, H:H + 1]
        pooled = sums / jnp.maximum(cnt, 1.0)
        hh = jnp.dot(pooled, lw1_ref[...],
                     preferred_element_type=jnp.float32) + lb1_ref[...]
        hh = jax.nn.relu(hh)
        hh = jnp.dot(hh, lw2_ref[...],
                     preferred_element_type=jnp.float32) + lb2_ref[...]
        m = jnp.max(hh, axis=-1, keepdims=True)
        lse = jnp.log(jnp.sum(jnp.exp(hh - m), axis=-1, keepdims=True)) + m
        out_ref[...] = hh - lse


def _tc4(agg, hs, dinv, b3, batch3, lw1, lb1, lw2, lb2):
    return pl.pallas_call(
        _tc4_body,
        grid=(GRID,),
        in_specs=[
            pl.BlockSpec((NC, BLK, 128), lambda i: (0, i, 0)),
            pl.BlockSpec((NC, BLK, 128), lambda i: (0, i, 0)),
            pl.BlockSpec((BLK, 1), lambda i: (i, 0)),
            pl.BlockSpec((1, H), lambda i: (0, 0)),
            pl.BlockSpec((1, 1, BLK), lambda i: (i, 0, 0)),
            pl.BlockSpec((H, H), lambda i: (0, 0)),
            pl.BlockSpec((1, H), lambda i: (0, 0)),
            pl.BlockSpec((H, C), lambda i: (0, 0)),
            pl.BlockSpec((1, C), lambda i: (0, 0)),
        ],
        out_specs=pl.BlockSpec((G, C), lambda i: (0, 0)),
        out_shape=jax.ShapeDtypeStruct((G, C), jnp.float32),
        scratch_shapes=[pltpu.VMEM((G, H + 128), jnp.float32)],
    )(agg, hs, dinv, b3, batch3, lw1, lb1, lw2, lb2)


def kernel(x, edge_index, edge_attr, batch, W1, b1, W2, b2, W3, b3,
           lw1, lb1, lw2, lb2):
    row = edge_index[0].astype(jnp.int32).reshape(NS, EPT)
    col = edge_index[1].astype(jnp.int32).reshape(NS, EPT)
    ew = edge_attr.astype(jnp.float32).reshape(NS, EPT)

    pad_i = jnp.zeros((NS, EPT_PAD - EPT), jnp.int32)
    pad_f = jnp.zeros((NS, EPT_PAD - EPT), jnp.float32)
    rowp = jnp.concatenate([row, pad_i], axis=1).reshape(NS, CH, 1, KE)
    colp = jnp.concatenate([col, pad_i], axis=1).reshape(NS, CH, 1, KE)
    ewp = jnp.concatenate([ew, pad_f], axis=1).reshape(NS, CH, KE)
    idx2 = jnp.stack(
        [jnp.concatenate([rowp, colp], axis=2),
         jnp.concatenate([rowp + N_PAD, colp], axis=2)], axis=0)

    xp = jnp.pad(x, ((0, N_PAD - N), (0, 0)))
    batchp = jnp.concatenate(
        [batch.astype(jnp.int32), jnp.full((N_PAD - N,), G, jnp.int32)])

    ones_tab = jnp.ones((NC * N_PAD, 128), jnp.float32)
    degp = _sc_agg(ones_tab, idx2, ewp)

    hs1, dinv = _tc1(xp, W1, degp)
    agg1 = _sc_agg(hs1.reshape(NC * N_PAD, 128), idx2, ewp).reshape(NC, N_PAD, 128)
    hs2 = _tc_mid(agg1, hs1, dinv, b1.reshape(1, H), W2)
    agg2 = _sc_agg(hs2.reshape(NC * N_PAD, 128), idx2, ewp).reshape(NC, N_PAD, 128)
    hs3 = _tc_mid(agg2, hs2, dinv, b2.reshape(1, H), W3)
    agg3 = _sc_agg(hs3.reshape(NC * N_PAD, 128), idx2, ewp).reshape(NC, N_PAD, 128)

    batch3 = batchp.reshape(GRID, 1, BLK)
    return _tc4(agg3, hs3, dinv, b3.reshape(1, H), batch3,
                lw1, lb1.reshape(1, H), lw2, lb2.reshape(1, C))
